# baseline (speedup 1.0000x reference)
import jax
import jax.numpy as jnp
from jax.experimental import pallas as pl
from jax.experimental.pallas import tpu as pltpu

N_WIDTH = 32
N_ORDER = 3
N_ELEMENTS = 16
N_NODES = N_ELEMENTS * N_ORDER + 1
N_SAMPLES = 2048
NDIM_IN = 4
DELTA_X = 0.5 * N_ORDER / (N_NODES - 1)

_NODES4 = [-1.0, -1.0 / 3.0, 1.0 / 3.0, 1.0]


def _lag_vals(xt):
    out = []
    for j in range(4):
        p = jnp.ones_like(xt)
        for m in range(4):
            if m != j:
                p = p * (xt - _NODES4[m]) / (_NODES4[j] - _NODES4[m])
        out.append(p)
    return out


def _dlag_vals(xt):
    out = []
    for j in range(4):
        y = jnp.zeros_like(xt)
        for i in range(4):
            if i != j:
                k = jnp.ones_like(xt) / (_NODES4[j] - _NODES4[i])
                for m in range(4):
                    if m != i and m != j:
                        k = k * (xt - _NODES4[m]) / (_NODES4[j] - _NODES4[m])
                y = y + k
        out.append(y)
    return out


def _ddlag_vals(xt):
    out = []
    for j in range(4):
        y = jnp.zeros_like(xt)
        for i in range(4):
            if i != j:
                k_sum = jnp.zeros_like(xt)
                for m in range(4):
                    if m != i and m != j:
                        k_prod = jnp.ones_like(xt) / (_NODES4[j] - _NODES4[m])
                        for n in range(4):
                            if n != i and n != j and n != m:
                                k_prod = k_prod * (xt - _NODES4[n]) / (_NODES4[j] - _NODES4[n])
                        k_sum = k_sum + k_prod
                y = y + k_sum / (_NODES4[j] - _NODES4[i])
        out.append(y)
    return out


def _local_coords(x_like):
    xs = float(N_NODES - 1) * x_like
    e = jnp.clip(jnp.floor(xs * (1.0 / N_ORDER)), 0.0, float(N_ELEMENTS - 1))
    nl = e * float(N_ORDER)
    xt = 2.0 * (xs - nl) * (1.0 / N_ORDER) - 1.0
    return e, xt


def _basis_triplet(xt):
    phi = _lag_vals(xt)
    dphi = [v * (1.0 / DELTA_X) for v in _dlag_vals(xt)]
    ddphi = [v * (1.0 / (DELTA_X * DELTA_X)) for v in _ddlag_vals(xt)]
    return phi, dphi, ddphi


def _body(x64_ref, xT_ref, wsplit_ref,
          t_ref, dt_ref, ddt_ref, oph_ref, odph_ref, oddph_ref,
          vph, vdph, vddph, sph, sdph, sddph):
    X = x64_ref[...]
    eA, xtA = _local_coords(X)
    eAi = eA.astype(jnp.int32)
    basisA = _basis_triplet(xtA)
    for vals, vref in zip(basisA, (vph, vdph, vddph)):
        for n in range(N_NODES):
            if n == N_NODES - 1:
                acc = jnp.where(eAi == N_ELEMENTS - 1, vals[3], 0.0)
            else:
                e1, p1 = n // 3, n % 3
                acc = jnp.where(eAi == e1, vals[p1], 0.0)
                if p1 == 0 and n > 0:
                    acc = acc + jnp.where(eAi == e1 - 1, vals[3], 0.0)
            vref[n] = acc

    copies = []
    for k in range(N_WIDTH):
        for vref, oref, sem in ((vph, oph_ref, sph), (vdph, odph_ref, sdph),
                                (vddph, oddph_ref, sddph)):
            c = pltpu.make_async_copy(vref, oref.at[k], sem)
            c.start()
            copies.append(c)

    xB = xT_ref[...]
    eB, xtB = _local_coords(xB)
    bB = (eB * float(N_ORDER)).astype(jnp.int32)
    basisB = _basis_triplet(xtB)
    n_iota = jax.lax.broadcasted_iota(jnp.int32, (N_NODES, N_SAMPLES), 0)
    for vals, tref in zip(basisB, (t_ref, dt_ref, ddt_ref)):
        acc_t = jnp.zeros((N_WIDTH, N_SAMPLES), jnp.float32)
        for j in range(NDIM_IN):
            bj = jnp.broadcast_to(bB[j:j + 1, :], (N_NODES, N_SAMPLES))
            S = jnp.zeros((N_NODES, N_SAMPLES), jnp.float32)
            for p in range(N_ORDER + 1):
                vj = jnp.broadcast_to(vals[p][j:j + 1, :], (N_NODES, N_SAMPLES))
                S = S + jnp.where(n_iota == bj + p, vj, 0.0)
            acc_t = acc_t + jnp.dot(wsplit_ref[j], S,
                                    preferred_element_type=jnp.float32)
        tref[...] = acc_t

    for c in copies:
        c.wait()


def kernel(x, weight):
    x64 = x.reshape(16, 128, 4).transpose(0, 2, 1).reshape(64, 128)
    xT = x.T
    wsplit = weight.transpose(2, 0, 1)

    tshape = jax.ShapeDtypeStruct((N_WIDTH, N_SAMPLES), jnp.float32)
    oshape = jax.ShapeDtypeStruct((N_WIDTH, N_NODES, 64, 128), jnp.float32)
    tspec = pl.BlockSpec((N_WIDTH, N_SAMPLES), lambda: (0, 0))
    ospec = pl.BlockSpec(memory_space=pltpu.MemorySpace.HBM)

    tT, dtT, ddtT, oph, odph, oddph = pl.pallas_call(
        _body,
        in_specs=[
            pl.BlockSpec((64, 128), lambda: (0, 0)),
            pl.BlockSpec((NDIM_IN, N_SAMPLES), lambda: (0, 0)),
            pl.BlockSpec((NDIM_IN, N_WIDTH, N_NODES), lambda: (0, 0, 0)),
        ],
        out_specs=[tspec, tspec, tspec, ospec, ospec, ospec],
        out_shape=(tshape, tshape, tshape, oshape, oshape, oshape),
        scratch_shapes=(
            [pltpu.VMEM((N_NODES, 64, 128), jnp.float32)] * 3
            + [pltpu.SemaphoreType.DMA] * 3
        ),
    )(x64, xT, wsplit)

    def _unpack(o):
        return (o.reshape(N_WIDTH, N_NODES, 16, 4, 128)
                 .transpose(2, 4, 0, 1, 3)
                 .reshape(N_SAMPLES, N_WIDTH, N_NODES, NDIM_IN))

    return {
        't_ik': tT.T, 'dt_ik': dtT.T, 'ddt_ik': ddtT.T,
        'phi_ikp': _unpack(oph),
        'dphi_ikp': _unpack(odph),
        'ddphi_ikp': _unpack(oddph),
        'delta_x': jnp.asarray(DELTA_X, jnp.float32),
    }


# speedup vs baseline: 1.0490x; 1.0490x over previous
import jax
import jax.numpy as jnp
from jax.experimental import pallas as pl
from jax.experimental.pallas import tpu as pltpu

N_WIDTH = 32
N_ORDER = 3
N_ELEMENTS = 16
N_NODES = N_ELEMENTS * N_ORDER + 1
N_SAMPLES = 2048
NDIM_IN = 4
DELTA_X = 0.5 * N_ORDER / (N_NODES - 1)

_NODES4 = [-1.0, -1.0 / 3.0, 1.0 / 3.0, 1.0]


def _lag_vals(xt):
    out = []
    for j in range(4):
        p = jnp.ones_like(xt)
        for m in range(4):
            if m != j:
                p = p * (xt - _NODES4[m]) / (_NODES4[j] - _NODES4[m])
        out.append(p)
    return out


def _dlag_vals(xt):
    out = []
    for j in range(4):
        y = jnp.zeros_like(xt)
        for i in range(4):
            if i != j:
                k = jnp.ones_like(xt) / (_NODES4[j] - _NODES4[i])
                for m in range(4):
                    if m != i and m != j:
                        k = k * (xt - _NODES4[m]) / (_NODES4[j] - _NODES4[m])
                y = y + k
        out.append(y)
    return out


def _ddlag_vals(xt):
    out = []
    for j in range(4):
        y = jnp.zeros_like(xt)
        for i in range(4):
            if i != j:
                k_sum = jnp.zeros_like(xt)
                for m in range(4):
                    if m != i and m != j:
                        k_prod = jnp.ones_like(xt) / (_NODES4[j] - _NODES4[m])
                        for n in range(4):
                            if n != i and n != j and n != m:
                                k_prod = k_prod * (xt - _NODES4[n]) / (_NODES4[j] - _NODES4[n])
                        k_sum = k_sum + k_prod
                y = y + k_sum / (_NODES4[j] - _NODES4[i])
        out.append(y)
    return out


def _local_coords(x_like):
    xs = float(N_NODES - 1) * x_like
    e = jnp.clip(jnp.floor(xs * (1.0 / N_ORDER)), 0.0, float(N_ELEMENTS - 1))
    nl = e * float(N_ORDER)
    xt = 2.0 * (xs - nl) * (1.0 / N_ORDER) - 1.0
    return e, xt


def _basis_triplet(xt):
    phi = _lag_vals(xt)
    dphi = [v * (1.0 / DELTA_X) for v in _dlag_vals(xt)]
    ddphi = [v * (1.0 / (DELTA_X * DELTA_X)) for v in _ddlag_vals(xt)]
    return phi, dphi, ddphi


_BK = 1


def _body(x64_ref, xT_ref, wsplit_ref,
          t_ref, dt_ref, ddt_ref, oph_ref, odph_ref, oddph_ref,
          vph, vdph, vddph):
    k = pl.program_id(0)

    @pl.when(k == 0)
    def _init():
        X = x64_ref[...]
        eA, xtA = _local_coords(X)
        eAi = eA.astype(jnp.int32)
        basisA = _basis_triplet(xtA)
        for vals, vref in zip(basisA, (vph, vdph, vddph)):
            for n in range(N_NODES):
                if n == N_NODES - 1:
                    acc = jnp.where(eAi == N_ELEMENTS - 1, vals[3], 0.0)
                else:
                    e1, p1 = n // 3, n % 3
                    acc = jnp.where(eAi == e1, vals[p1], 0.0)
                    if p1 == 0 and n > 0:
                        acc = acc + jnp.where(eAi == e1 - 1, vals[3], 0.0)
                vref[n] = acc

    @pl.when(k == pl.num_programs(0) - 1)
    def _reduce():
        xB = xT_ref[...]
        eB, xtB = _local_coords(xB)
        bB = (eB * float(N_ORDER)).astype(jnp.int32)
        basisB = _basis_triplet(xtB)
        n_iota = jax.lax.broadcasted_iota(jnp.int32, (N_NODES, N_SAMPLES), 0)
        for vals, tref in zip(basisB, (t_ref, dt_ref, ddt_ref)):
            acc_t = jnp.zeros((N_WIDTH, N_SAMPLES), jnp.float32)
            for j in range(NDIM_IN):
                bj = jnp.broadcast_to(bB[j:j + 1, :], (N_NODES, N_SAMPLES))
                S = jnp.zeros((N_NODES, N_SAMPLES), jnp.float32)
                for p in range(N_ORDER + 1):
                    vj = jnp.broadcast_to(vals[p][j:j + 1, :], (N_NODES, N_SAMPLES))
                    S = S + jnp.where(n_iota == bj + p, vj, 0.0)
                acc_t = acc_t + jnp.dot(wsplit_ref[j], S,
                                        preferred_element_type=jnp.float32)
            tref[...] = acc_t

    for kk in range(_BK):
        oph_ref[kk] = vph[...]
        odph_ref[kk] = vdph[...]
        oddph_ref[kk] = vddph[...]


def kernel(x, weight):
    x64 = x.reshape(16, 128, 4).transpose(0, 2, 1).reshape(64, 128)
    xT = x.T
    wsplit = weight.transpose(2, 0, 1)

    tshape = jax.ShapeDtypeStruct((N_WIDTH, N_SAMPLES), jnp.float32)
    oshape = jax.ShapeDtypeStruct((N_WIDTH, N_NODES, 64, 128), jnp.float32)
    tspec = pl.BlockSpec((N_WIDTH, N_SAMPLES), lambda k: (0, 0))
    ospec = pl.BlockSpec((_BK, N_NODES, 64, 128), lambda k: (k, 0, 0, 0))

    tT, dtT, ddtT, oph, odph, oddph = pl.pallas_call(
        _body,
        grid=(N_WIDTH // _BK,),
        in_specs=[
            pl.BlockSpec((64, 128), lambda k: (0, 0)),
            pl.BlockSpec((NDIM_IN, N_SAMPLES), lambda k: (0, 0)),
            pl.BlockSpec((NDIM_IN, N_WIDTH, N_NODES), lambda k: (0, 0, 0)),
        ],
        out_specs=[tspec, tspec, tspec, ospec, ospec, ospec],
        out_shape=(tshape, tshape, tshape, oshape, oshape, oshape),
        scratch_shapes=[pltpu.VMEM((N_NODES, 64, 128), jnp.float32)] * 3,
        compiler_params=pltpu.CompilerParams(
            dimension_semantics=("arbitrary",)),
    )(x64, xT, wsplit)

    def _unpack(o):
        return (o.reshape(N_WIDTH, N_NODES, 16, 4, 128)
                 .transpose(2, 4, 0, 1, 3)
                 .reshape(N_SAMPLES, N_WIDTH, N_NODES, NDIM_IN))

    return {
        't_ik': tT.T, 'dt_ik': dtT.T, 'ddt_ik': ddtT.T,
        'phi_ikp': _unpack(oph),
        'dphi_ikp': _unpack(odph),
        'ddphi_ikp': _unpack(oddph),
        'delta_x': jnp.asarray(DELTA_X, jnp.float32),
    }
